# Initial kernel scaffold; baseline (speedup 1.0000x reference)
import jax
import jax.numpy as jnp
from jax.experimental import pallas as pl


def kernel(hidden_states, router_W, router_b, expert_W, expert_b):
    raise NotImplementedError("write your pallas kernel here")


# speedup vs baseline: 66.8802x; 66.8802x over previous
import functools

import jax
import jax.numpy as jnp
from jax.experimental import pallas as pl
from jax.experimental.pallas import tpu as pltpu

E = 8
D = 768
F = 768
TBLK = 256


def _moe_body(h_ref, wr_ref, rb_ref, we_ref, eb_ref, out_ref):
    h = h_ref[...]
    logits = jnp.dot(h, wr_ref[...], preferred_element_type=jnp.float32)
    logits = logits + rb_ref[...]
    chunks = [logits[:, e * F:(e + 1) * F] for e in range(E)]

    m1 = jnp.maximum(chunks[0], chunks[1])
    m2 = jnp.minimum(chunks[0], chunks[1])
    for e in range(2, E):
        x = chunks[e]
        m2 = jnp.maximum(m2, jnp.minimum(m1, x))
        m1 = jnp.maximum(m1, x)

    t = jnp.exp(m2 - m1)
    w1 = 1.0 / (1.0 + t)
    w2 = 1.0 - w1

    wt_cols = []
    for e in range(E):
        x = chunks[e]
        mass = jnp.where(x == m1, w1, jnp.where(x == m2, w2, 0.0))
        wt_cols.append(jnp.sum(mass, axis=1, keepdims=True))

    wt = jnp.concatenate(wt_cols, axis=1)
    acc = jnp.dot(wt, eb_ref[...], preferred_element_type=jnp.float32)
    for e in range(E):
        pe = jnp.dot(h, we_ref[e], preferred_element_type=jnp.float32)
        acc = acc + pe * wt_cols[e]
    out_ref[...] = acc


@jax.jit
def _run(h, wr, rb, we, eb):
    T = h.shape[0]
    grid = (T // TBLK,)
    return pl.pallas_call(
        _moe_body,
        grid=grid,
        in_specs=[
            pl.BlockSpec((TBLK, D), lambda i: (i, 0)),
            pl.BlockSpec((D, E * F), lambda i: (0, 0)),
            pl.BlockSpec((1, E * F), lambda i: (0, 0)),
            pl.BlockSpec((E, D, F), lambda i: (0, 0, 0)),
            pl.BlockSpec((E, F), lambda i: (0, 0)),
        ],
        out_specs=pl.BlockSpec((TBLK, F), lambda i: (i, 0)),
        out_shape=jax.ShapeDtypeStruct((T, F), jnp.float32),
        compiler_params=pltpu.CompilerParams(
            dimension_semantics=("parallel",),
        ),
    )(h, wr, rb, we, eb)


def kernel(hidden_states, router_W, router_b, expert_W, expert_b):
    B, S, Dh = hidden_states.shape
    T = B * S
    h = hidden_states.reshape(T, Dh).astype(jnp.bfloat16)
    wr = router_W.reshape(Dh, F, E).transpose(0, 2, 1).reshape(Dh, E * F)
    wr = wr.astype(jnp.bfloat16)
    rb = router_b.reshape(F, E).T.reshape(1, E * F).astype(jnp.float32)
    we = expert_W.astype(jnp.bfloat16)
    eb = expert_b.astype(jnp.float32)
    out = _run(h, wr, rb, we, eb)
    return out.reshape(B, S, F)


# speedup vs baseline: 68.3946x; 1.0226x over previous
import functools

import jax
import jax.numpy as jnp
from jax.experimental import pallas as pl
from jax.experimental.pallas import tpu as pltpu

E = 8
D = 768
F = 768
TBLK = 256


def _moe_body(h_ref, wr_ref, rb_ref, we_ref, eb_ref, out_ref):
    h = h_ref[...].astype(jnp.bfloat16)
    logits = jnp.dot(h, wr_ref[...], preferred_element_type=jnp.float32)
    logits = logits + rb_ref[...]
    chunks = [logits[:, e * F:(e + 1) * F] for e in range(E)]

    m1 = jnp.maximum(chunks[0], chunks[1])
    m2 = jnp.minimum(chunks[0], chunks[1])
    for e in range(2, E):
        x = chunks[e]
        m2 = jnp.maximum(m2, jnp.minimum(m1, x))
        m1 = jnp.maximum(m1, x)

    t = jnp.exp(m2 - m1)
    w1 = 1.0 / (1.0 + t)
    w2 = 1.0 - w1

    wt_cols = []
    for e in range(E):
        x = chunks[e]
        mass = jnp.where(x == m1, w1, jnp.where(x == m2, w2, 0.0))
        wt_cols.append(jnp.sum(mass, axis=1, keepdims=True))

    wt = jnp.concatenate(wt_cols, axis=1)
    acc = jnp.dot(wt, eb_ref[...], preferred_element_type=jnp.float32)
    for e in range(E):
        pe = jnp.dot(h, we_ref[e].astype(jnp.bfloat16),
                     preferred_element_type=jnp.float32)
        acc = acc + pe * wt_cols[e]
    out_ref[...] = acc


@jax.jit
def _run(h, wr, rb, we, eb):
    T = h.shape[0]
    grid = (T // TBLK,)
    return pl.pallas_call(
        _moe_body,
        grid=grid,
        in_specs=[
            pl.BlockSpec((TBLK, D), lambda i: (i, 0)),
            pl.BlockSpec((D, E * F), lambda i: (0, 0)),
            pl.BlockSpec((1, E * F), lambda i: (0, 0)),
            pl.BlockSpec((E, D, F), lambda i: (0, 0, 0)),
            pl.BlockSpec((E, F), lambda i: (0, 0)),
        ],
        out_specs=pl.BlockSpec((TBLK, F), lambda i: (i, 0)),
        out_shape=jax.ShapeDtypeStruct((T, F), jnp.float32),
        compiler_params=pltpu.CompilerParams(
            dimension_semantics=("parallel",),
        ),
    )(h, wr, rb, we, eb)


def kernel(hidden_states, router_W, router_b, expert_W, expert_b):
    B, S, Dh = hidden_states.shape
    T = B * S
    h = hidden_states.reshape(T, Dh)
    wr = router_W.reshape(Dh, F, E).transpose(0, 2, 1).reshape(Dh, E * F)
    wr = wr.astype(jnp.bfloat16)
    rb = router_b.reshape(F, E).T.reshape(1, E * F).astype(jnp.float32)
    we = expert_W
    eb = expert_b.astype(jnp.float32)
    out = _run(h, wr, rb, we, eb)
    return out.reshape(B, S, F)
